# baseline (speedup 1.0000x reference)
import functools

import jax
import jax.numpy as jnp
from jax.experimental import pallas as pl
from jax.experimental.pallas import tpu as pltpu

_EPS = 1e-6


def _fused_body(x_ref, g_ref, b_ref, w_ref, wb_ref, o_ref,
                wg_ref, cst_ref, gb_ref, xb_ref, *, tm, ck, inv_d, inv_dm1):
    j = pl.program_id(1)

    @pl.when(j == 0)
    def _build_constants():
        g = g_ref[...]
        g_col = jnp.transpose(g)
        wg_ref[...] = (g_col * w_ref[...]).astype(jnp.bfloat16)
        gb_ref[0:1, :] = g
        gb_ref[1:2, :] = b_ref[...]
        small = jnp.dot(gb_ref[...], w_ref[...],
                        preferred_element_type=jnp.float32)
        cst_ref[0:1, :] = small[0:1, :]
        cst_ref[1:2, :] = small[1:2, :] + wb_ref[...]

    xb_ref[...] = x_ref[...].astype(jnp.bfloat16)
    xb = xb_ref[...]
    s1 = jnp.sum(xb, axis=-1, keepdims=True, dtype=jnp.float32)
    xf = xb.astype(jnp.float32)
    s2 = jnp.sum(xf * xf, axis=-1, keepdims=True)
    mean = s1 * inv_d
    var = (s2 - s1 * mean) * inv_dm1
    inv = 1.0 / (jnp.sqrt(var) + _EPS)
    c = inv * mean

    gw = cst_ref[0:1, :]
    add_row = cst_ref[1:2, :]
    for k in range(tm // ck):
        sl = slice(k * ck, (k + 1) * ck)
        xw = jnp.dot(xb_ref[sl, :], wg_ref[...],
                     preferred_element_type=jnp.float32)
        o_ref[sl, :] = (xb_ref[sl, :].astype(jnp.float32) + inv[sl, :] * xw
                        - c[sl, :] * gw + add_row)


def _pick_tm(m):
    for t in (1024, 512, 256, 128, 64, 32, 16, 8):
        if m % t == 0:
            return t
    return 8


def kernel(x, a_2, b_2, w, wb):
    B, S, D = x.shape
    M = B * S

    g2 = a_2.reshape(1, D)
    b2 = b_2.reshape(1, D)
    wb2 = wb.reshape(1, D)

    x2 = x.reshape(M, D)
    tm = _pick_tm(M)
    ck = min(tm, 256)
    m_pad = ((M + tm - 1) // tm) * tm
    if m_pad != M:
        x2 = jnp.pad(x2, ((0, m_pad - M), (0, 0)))

    nt = m_pad // tm
    ni = 2 if nt % 2 == 0 else 1
    nj = nt // ni

    out2 = pl.pallas_call(
        functools.partial(_fused_body, tm=tm, ck=ck, inv_d=1.0 / D,
                          inv_dm1=1.0 / (D - 1)),
        out_shape=jax.ShapeDtypeStruct((m_pad, D), x.dtype),
        grid=(ni, nj),
        in_specs=[
            pl.BlockSpec((tm, D), lambda i, j, nj=nj: (i * nj + j, 0)),
            pl.BlockSpec((1, D), lambda i, j: (0, 0),
                         pipeline_mode=pl.Buffered(1)),
            pl.BlockSpec((1, D), lambda i, j: (0, 0),
                         pipeline_mode=pl.Buffered(1)),
            pl.BlockSpec((D, D), lambda i, j: (0, 0),
                         pipeline_mode=pl.Buffered(1)),
            pl.BlockSpec((1, D), lambda i, j: (0, 0),
                         pipeline_mode=pl.Buffered(1)),
        ],
        out_specs=pl.BlockSpec((tm, D), lambda i, j, nj=nj: (i * nj + j, 0)),
        scratch_shapes=[pltpu.VMEM((D, D), jnp.bfloat16),
                        pltpu.VMEM((8, D), jnp.float32),
                        pltpu.VMEM((8, D), jnp.float32),
                        pltpu.VMEM((tm, D), jnp.bfloat16)],
        compiler_params=pltpu.CompilerParams(
            dimension_semantics=("parallel", "arbitrary"),
            vmem_limit_bytes=64 << 20,
        ),
    )(x2, g2, b2, w, wb2)

    return out2[:M].reshape(B, S, D)


# speedup vs baseline: 1.0934x; 1.0934x over previous
import functools

import jax
import jax.numpy as jnp
from jax.experimental import pallas as pl
from jax.experimental.pallas import tpu as pltpu

_EPS = 1e-6


def _fused_body(x_ref, g_ref, b_ref, w_ref, wb_ref, o_ref,
                wg_ref, cst_ref, gb_ref, xb_ref, *, tm, ck, inv_d, inv_dm1):
    j = pl.program_id(1)

    @pl.when(j == 0)
    def _build_constants():
        g = g_ref[...]
        g_col = jnp.transpose(g)
        wg_ref[...] = (g_col * w_ref[...]).astype(jnp.bfloat16)
        gb_ref[0:1, :] = g
        gb_ref[1:2, :] = b_ref[...]
        small = jnp.dot(gb_ref[...], w_ref[...],
                        preferred_element_type=jnp.float32)
        cst_ref[0:1, :] = small[0:1, :]
        cst_ref[1:2, :] = small[1:2, :] + wb_ref[...]

    x = x_ref[...]
    xb_ref[...] = x.astype(jnp.bfloat16)
    s1 = jnp.sum(x, axis=-1, keepdims=True)
    s2 = jnp.sum(x * x, axis=-1, keepdims=True)
    mean = s1 * inv_d
    var = (s2 - s1 * mean) * inv_dm1
    inv = 1.0 / (jnp.sqrt(var) + _EPS)
    c = inv * mean

    gw = cst_ref[0:1, :]
    add_row = cst_ref[1:2, :]
    for k in range(tm // ck):
        sl = slice(k * ck, (k + 1) * ck)
        xw = jnp.dot(xb_ref[sl, :], wg_ref[...],
                     preferred_element_type=jnp.float32)
        o_ref[sl, :] = (x_ref[sl, :] + inv[sl, :] * xw
                        - c[sl, :] * gw + add_row)


def _pick_tm(m):
    for t in (2048, 1024, 512, 256, 128, 64, 32, 16, 8):
        if m % t == 0:
            return t
    return 8


def kernel(x, a_2, b_2, w, wb):
    B, S, D = x.shape
    M = B * S

    g2 = a_2.reshape(1, D)
    b2 = b_2.reshape(1, D)
    wb2 = wb.reshape(1, D)

    x2 = x.reshape(M, D)
    tm = _pick_tm(M)
    ck = min(tm, 256)
    m_pad = ((M + tm - 1) // tm) * tm
    if m_pad != M:
        x2 = jnp.pad(x2, ((0, m_pad - M), (0, 0)))

    nt = m_pad // tm
    ni = 2 if nt % 2 == 0 else 1
    nj = nt // ni

    out2 = pl.pallas_call(
        functools.partial(_fused_body, tm=tm, ck=ck, inv_d=1.0 / D,
                          inv_dm1=1.0 / (D - 1)),
        out_shape=jax.ShapeDtypeStruct((m_pad, D), x.dtype),
        grid=(ni, nj),
        in_specs=[
            pl.BlockSpec((tm, D), lambda i, j, nj=nj: (i * nj + j, 0)),
            pl.BlockSpec((1, D), lambda i, j: (0, 0),
                         pipeline_mode=pl.Buffered(1)),
            pl.BlockSpec((1, D), lambda i, j: (0, 0),
                         pipeline_mode=pl.Buffered(1)),
            pl.BlockSpec((D, D), lambda i, j: (0, 0),
                         pipeline_mode=pl.Buffered(1)),
            pl.BlockSpec((1, D), lambda i, j: (0, 0),
                         pipeline_mode=pl.Buffered(1)),
        ],
        out_specs=pl.BlockSpec((tm, D), lambda i, j, nj=nj: (i * nj + j, 0)),
        scratch_shapes=[pltpu.VMEM((D, D), jnp.bfloat16),
                        pltpu.VMEM((8, D), jnp.float32),
                        pltpu.VMEM((8, D), jnp.float32),
                        pltpu.VMEM((tm, D), jnp.bfloat16)],
        compiler_params=pltpu.CompilerParams(
            dimension_semantics=("parallel", "arbitrary"),
            vmem_limit_bytes=64 << 20,
        ),
    )(x2, g2, b2, w, wb2)

    return out2[:M].reshape(B, S, D)
